# baseline (speedup 1.0000x reference)
import functools

import jax
import jax.numpy as jnp
from jax import lax
from jax.experimental import pallas as pl
from jax.experimental.pallas import tpu as pltpu
from jax.experimental.pallas import tpu_sc as plsc

VOCAB = 100000
D = 100
DP = 128
L = 20
B = 16384
HIDDEN = 4096

CEX = 16
ROWS_PER_CHUNK = CEX * L
GB = 128
IDX_ROWS = (2 * B * L) // GB


def _idx_prep_body(p_ref, h_ref, o_ref):
    o_ref[...] = jnp.concatenate([p_ref[...], h_ref[...]], axis=0)


def _idx_prep(premise_t, hyp_t):
    return pl.pallas_call(
        _idx_prep_body,
        grid=(B // GB,),
        in_specs=[
            pl.BlockSpec((L, GB), lambda i: (0, i)),
            pl.BlockSpec((L, GB), lambda i: (0, i)),
        ],
        out_specs=pl.BlockSpec((2 * L, GB), lambda i: (i, 0)),
        out_shape=jax.ShapeDtypeStruct((IDX_ROWS, GB), jnp.int32),
    )(premise_t, hyp_t)


def _make_sc_pool():
    info = plsc.get_sparse_core_info()
    nc, ns = info.num_cores, info.num_subcores
    nw = nc * ns
    ex_per_w = B // nw
    groups_per_w = ex_per_w // GB
    idx_rows_per_w = groups_per_w * 2 * L
    chunks_per_half = ex_per_w // CEX
    pairs = chunks_per_half // 2

    mesh = plsc.VectorSubcoreMesh(core_axis_name="c", subcore_axis_name="s")

    @functools.partial(
        pl.kernel,
        mesh=mesh,
        out_type=(
            jax.ShapeDtypeStruct((B, DP), jnp.float32),
            jax.ShapeDtypeStruct((B, DP), jnp.float32),
        ),
        scratch_types=[
            pltpu.VMEM((idx_rows_per_w, GB), jnp.int32),
            pltpu.VMEM((ROWS_PER_CHUNK, DP), jnp.float32),
            pltpu.VMEM((ROWS_PER_CHUNK, DP), jnp.float32),
            pltpu.VMEM((CEX, DP), jnp.float32),
            pltpu.VMEM((CEX, DP), jnp.float32),
            pltpu.SemaphoreType.DMA,
            pltpu.SemaphoreType.DMA,
            pltpu.SemaphoreType.DMA,
            pltpu.SemaphoreType.DMA,
        ],
    )
    def pool_kernel(table_hbm, idx_hbm, prem_hbm, hyp_hbm, idx_all,
                    rows_v0, rows_v1, out_v0, out_v1,
                    sem0, sem1, osem0, osem1):
        wid = lax.axis_index("s") * nc + lax.axis_index("c")
        b0_w = wid * ex_per_w
        row_bufs = (rows_v0, rows_v1)
        out_bufs = (out_v0, out_v1)
        sems = (sem0, sem1)
        osems = (osem0, osem1)
        out_hbms = (prem_hbm, hyp_hbm)

        pltpu.sync_copy(
            idx_hbm.at[pl.ds(wid * idx_rows_per_w, idx_rows_per_w)], idx_all)

        def issue(buf, half, c):
            g128 = c // (GB // CEX)
            col0 = (c % (GB // CEX)) * CEX
            rbase = g128 * 2 * L + half * L
            for l in range(L):
                pltpu.async_copy(
                    table_hbm.at[idx_all.at[rbase + l, pl.ds(col0, CEX)]],
                    row_bufs[buf].at[pl.ds(l * CEX, CEX)],
                    sems[buf],
                )

        def wait_buf(buf):
            pltpu.make_async_copy(
                table_hbm.at[pl.ds(0, ROWS_PER_CHUNK)],
                row_bufs[buf],
                sems[buf],
            ).wait()

        def wait_out(buf):
            pltpu.make_async_copy(
                out_bufs[buf],
                prem_hbm.at[pl.ds(0, CEX)],
                osems[buf],
            ).wait()

        def compute_store(buf, half, c, have_outstanding):
            rows_v = row_bufs[buf]
            out_v = out_bufs[buf]

            @pl.when(have_outstanding)
            def _():
                wait_out(buf)

            def ex_body(j, carry2):
                for d in range(DP // 16):
                    sl = pl.ds(d * 16, 16)
                    acc = rows_v[j, sl]
                    for l in range(1, L):
                        acc = jnp.maximum(acc, rows_v[l * CEX + j, sl])
                    out_v[j, sl] = acc
                return carry2

            lax.fori_loop(0, CEX, ex_body, 0, unroll=False)
            pltpu.async_copy(
                out_v,
                out_hbms[half].at[pl.ds(b0_w + c * CEX, CEX)],
                osems[buf],
            )

        issue(0, 0, 0)
        for half in (0, 1):

            def pair_body(t, carry, half=half):
                c0 = 2 * t
                first = 0 if half == 0 else -1
                issue(1, half, c0 + 1)
                wait_buf(0)
                compute_store(0, half, c0, t > first)

                @pl.when(t < pairs - 1)
                def _():
                    issue(0, half, c0 + 2)

                wait_buf(1)
                compute_store(1, half, c0 + 1, t > first)
                return carry

            lax.fori_loop(0, pairs, pair_body, 0, unroll=False)
            if half == 0:
                issue(0, 1, 0)

        wait_out(0)
        wait_out(1)

    return pool_kernel


_TR_BV = 2048
_TR_GRID = (VOCAB + _TR_BV - 1) // _TR_BV


def _transpose_body(xt_ref, o_ref):
    x = xt_ref[...]
    xp = jnp.concatenate(
        [x, jnp.zeros((DP - D, _TR_BV), jnp.float32)], axis=0)
    r = lax.broadcasted_iota(jnp.int32, (DP, DP), 0)
    c = lax.broadcasted_iota(jnp.int32, (DP, DP), 1)
    eye = jnp.where(r == c, 1.0, 0.0).astype(jnp.float32)
    o_ref[...] = lax.dot_general(
        xp, eye, (((0,), (0,)), ((), ())),
        preferred_element_type=jnp.float32)


def _transpose_pad(emb_t):
    return pl.pallas_call(
        _transpose_body,
        grid=(_TR_GRID,),
        in_specs=[pl.BlockSpec((D, _TR_BV), lambda i: (0, i))],
        out_specs=pl.BlockSpec((_TR_BV, DP), lambda i: (i, 0)),
        out_shape=jax.ShapeDtypeStruct((VOCAB, DP), jnp.float32),
    )(emb_t)


_TC_BM = 512


def _mlp_body(xp_ref, xh_ref, w1a_ref, w1b_ref, b1_ref, w2_ref, b2_ref, o_ref):
    xp = xp_ref[...].astype(jnp.bfloat16)
    xh = xh_ref[...].astype(jnp.bfloat16)
    h = jnp.dot(xp, w1a_ref[...], preferred_element_type=jnp.float32)
    h = h + jnp.dot(xh, w1b_ref[...], preferred_element_type=jnp.float32)
    h = jnp.maximum(h + b1_ref[...], 0.0)
    y = jnp.sum(h * w2_ref[...], axis=1) + b2_ref[0]
    o_ref[...] = jax.nn.sigmoid(y)


def _mlp(prem, hyp, w1a, w1b, b1, w2, b2):
    grid = (B // _TC_BM,)
    return pl.pallas_call(
        _mlp_body,
        grid=grid,
        in_specs=[
            pl.BlockSpec((_TC_BM, DP), lambda i: (i, 0)),
            pl.BlockSpec((_TC_BM, DP), lambda i: (i, 0)),
            pl.BlockSpec((DP, HIDDEN), lambda i: (0, 0)),
            pl.BlockSpec((DP, HIDDEN), lambda i: (0, 0)),
            pl.BlockSpec((1, HIDDEN), lambda i: (0, 0)),
            pl.BlockSpec((1, HIDDEN), lambda i: (0, 0)),
            pl.BlockSpec(memory_space=pltpu.SMEM),
        ],
        out_specs=pl.BlockSpec((_TC_BM,), lambda i: (i,)),
        out_shape=jax.ShapeDtypeStruct((B,), jnp.float32),
    )(prem, hyp, w1a, w1b, b1.reshape(1, HIDDEN), w2.reshape(1, HIDDEN), b2)


def kernel(premise, hypothesis, emb_table, W1, b1, W2, b2):
    pool_kernel = _make_sc_pool()

    emb_p = _transpose_pad(emb_table.T)
    idx2d = _idx_prep(premise.T, hypothesis.T)

    prem_f, hyp_f = pool_kernel(emb_p, idx2d)

    zpad = jnp.zeros((DP - D, HIDDEN), dtype=W1.dtype)
    w1a = jnp.concatenate([W1[:D], zpad], axis=0).astype(jnp.bfloat16)
    w1b = jnp.concatenate([W1[D:], zpad], axis=0).astype(jnp.bfloat16)

    return _mlp(prem_f, hyp_f, w1a, w1b, b1, W2, b2)


# speedup vs baseline: 1.1267x; 1.1267x over previous
import functools

import jax
import jax.numpy as jnp
from jax import lax
from jax.experimental import pallas as pl
from jax.experimental.pallas import tpu as pltpu
from jax.experimental.pallas import tpu_sc as plsc

VOCAB = 100000
D = 100
DP = 128
L = 20
B = 16384
NEX = 2 * B
HIDDEN = 4096

CEX = 16
ROWS_PER_CHUNK = CEX * L
GATHERS = ((0, 128), (128, 128), (256, 64))


NSLICE = 4


def _make_sc_pool(nex_s):
    info = plsc.get_sparse_core_info()
    nc, ns = info.num_cores, info.num_subcores
    nw = nc * ns
    chunks_per_w = nex_s // nw // CEX
    pairs = chunks_per_w // 2
    idx_per_w = chunks_per_w * ROWS_PER_CHUNK

    mesh = plsc.VectorSubcoreMesh(core_axis_name="c", subcore_axis_name="s")

    @functools.partial(
        pl.kernel,
        mesh=mesh,
        out_type=(
            jax.ShapeDtypeStruct((nex_s // 2, DP), jnp.float32),
            jax.ShapeDtypeStruct((nex_s // 2, DP), jnp.float32),
        ),
        scratch_types=[
            pltpu.VMEM((idx_per_w,), jnp.int32),
            pltpu.VMEM((ROWS_PER_CHUNK, DP), jnp.float32),
            pltpu.VMEM((ROWS_PER_CHUNK, DP), jnp.float32),
            pltpu.VMEM((CEX // 2, DP), jnp.float32),
            pltpu.VMEM((CEX // 2, DP), jnp.float32),
            pltpu.VMEM((CEX // 2, DP), jnp.float32),
            pltpu.VMEM((CEX // 2, DP), jnp.float32),
            pltpu.SemaphoreType.DMA,
            pltpu.SemaphoreType.DMA,
            pltpu.SemaphoreType.DMA,
            pltpu.SemaphoreType.DMA,
        ],
    )
    def pool_kernel(table_hbm, idx_hbm, prem_hbm, hyp_hbm, idx_all,
                    rows_v0, rows_v1,
                    out_p0, out_p1, out_h0, out_h1,
                    sem0, sem1, osem0, osem1):
        wid = lax.axis_index("s") * nc + lax.axis_index("c")
        chunk0 = wid * chunks_per_w
        row_bufs = (rows_v0, rows_v1)
        outp_bufs = (out_p0, out_p1)
        outh_bufs = (out_h0, out_h1)
        sems = (sem0, sem1)
        osems = (osem0, osem1)

        pltpu.sync_copy(idx_hbm.at[pl.ds(wid * idx_per_w, idx_per_w)], idx_all)

        def issue(buf, c_local):
            base = c_local * ROWS_PER_CHUNK
            for off, gs in GATHERS:
                pltpu.async_copy(
                    table_hbm.at[idx_all.at[pl.ds(base + off, gs)]],
                    row_bufs[buf].at[pl.ds(off, gs)],
                    sems[buf],
                )

        def wait_buf(buf):
            pltpu.make_async_copy(
                table_hbm.at[pl.ds(0, ROWS_PER_CHUNK)],
                row_bufs[buf],
                sems[buf],
            ).wait()

        def wait_out(buf):
            pltpu.make_async_copy(
                outp_bufs[buf],
                prem_hbm.at[pl.ds(0, CEX // 2)],
                osems[buf],
            ).wait()
            pltpu.make_async_copy(
                outh_bufs[buf],
                prem_hbm.at[pl.ds(0, CEX // 2)],
                osems[buf],
            ).wait()

        def compute_store(buf, c_local, have_outstanding):
            rows_v = row_bufs[buf]
            out_p = outp_bufs[buf]
            out_h = outh_bufs[buf]

            @pl.when(have_outstanding)
            def _():
                wait_out(buf)

            def pair_ex_body(j, carry2):
                r0 = j * 2 * L
                for d in range(DP // 16):
                    sl = pl.ds(d * 16, 16)
                    acc = rows_v[r0, sl]
                    for l in range(1, L):
                        acc = jnp.maximum(acc, rows_v[r0 + l, sl])
                    out_p[j, sl] = acc
                    acch = rows_v[r0 + L, sl]
                    for l in range(1, L):
                        acch = jnp.maximum(acch, rows_v[r0 + L + l, sl])
                    out_h[j, sl] = acch
                return carry2

            lax.fori_loop(0, CEX // 2, pair_ex_body, 0, unroll=False)
            b0 = (chunk0 + c_local) * (CEX // 2)
            pltpu.async_copy(
                out_p, prem_hbm.at[pl.ds(b0, CEX // 2)], osems[buf])
            pltpu.async_copy(
                out_h, hyp_hbm.at[pl.ds(b0, CEX // 2)], osems[buf])

        issue(0, 0)

        def pair_body(t, carry):
            c0 = 2 * t
            issue(1, c0 + 1)
            wait_buf(0)
            compute_store(0, c0, t > 0)

            @pl.when(t < pairs - 1)
            def _():
                issue(0, c0 + 2)

            wait_buf(1)
            compute_store(1, c0 + 1, t > 0)
            return carry

        lax.fori_loop(0, pairs, pair_body, 0, unroll=False)
        wait_out(0)
        wait_out(1)

    return pool_kernel


_TR_BV = 2048
_TR_GRID = (VOCAB + _TR_BV - 1) // _TR_BV


def _transpose_body(xt_ref, o_ref):
    x = xt_ref[...]
    xp = jnp.concatenate(
        [x, jnp.zeros((DP - D, _TR_BV), jnp.float32)], axis=0)
    r = lax.broadcasted_iota(jnp.int32, (DP, DP), 0)
    c = lax.broadcasted_iota(jnp.int32, (DP, DP), 1)
    eye = jnp.where(r == c, 1.0, 0.0).astype(jnp.float32)
    o_ref[...] = lax.dot_general(
        xp, eye, (((0,), (0,)), ((), ())),
        preferred_element_type=jnp.float32)


def _transpose_pad(emb_t):
    return pl.pallas_call(
        _transpose_body,
        grid=(_TR_GRID,),
        in_specs=[pl.BlockSpec((D, _TR_BV), lambda i: (0, i))],
        out_specs=pl.BlockSpec((_TR_BV, DP), lambda i: (i, 0)),
        out_shape=jax.ShapeDtypeStruct((VOCAB, DP), jnp.float32),
    )(emb_t)


_TC_BM = 512


def _mlp_body(xp_ref, xh_ref, w1a_ref, w1b_ref, b1_ref, w2_ref, b2_ref, o_ref):
    xp = xp_ref[...].astype(jnp.bfloat16)
    xh = xh_ref[...].astype(jnp.bfloat16)
    h = jnp.dot(xp, w1a_ref[...], preferred_element_type=jnp.float32)
    h = h + jnp.dot(xh, w1b_ref[...], preferred_element_type=jnp.float32)
    h = jnp.maximum(h + b1_ref[...], 0.0)
    y = jnp.sum(h * w2_ref[...], axis=1) + b2_ref[0]
    o_ref[...] = jax.nn.sigmoid(y)


def _mlp(prem, hyp, w1a, w1b, b1, w2, b2):
    bs = prem.shape[0]
    grid = (bs // _TC_BM,)
    return pl.pallas_call(
        _mlp_body,
        grid=grid,
        in_specs=[
            pl.BlockSpec((_TC_BM, DP), lambda i: (i, 0)),
            pl.BlockSpec((_TC_BM, DP), lambda i: (i, 0)),
            pl.BlockSpec((DP, HIDDEN), lambda i: (0, 0)),
            pl.BlockSpec((DP, HIDDEN), lambda i: (0, 0)),
            pl.BlockSpec((1, HIDDEN), lambda i: (0, 0)),
            pl.BlockSpec((1, HIDDEN), lambda i: (0, 0)),
            pl.BlockSpec(memory_space=pltpu.SMEM),
        ],
        out_specs=pl.BlockSpec((_TC_BM,), lambda i: (i,)),
        out_shape=jax.ShapeDtypeStruct((bs,), jnp.float32),
    )(prem, hyp, w1a, w1b, b1.reshape(1, HIDDEN), w2.reshape(1, HIDDEN), b2)


def kernel(premise, hypothesis, emb_table, W1, b1, W2, b2):
    nex_s = NEX // NSLICE
    pool_kernel = _make_sc_pool(nex_s)

    emb_p = _transpose_pad(emb_table.T)
    idx = jnp.stack([premise, hypothesis], axis=1).reshape(-1)

    zpad = jnp.zeros((DP - D, HIDDEN), dtype=W1.dtype)
    w1a = jnp.concatenate([W1[:D], zpad], axis=0).astype(jnp.bfloat16)
    w1b = jnp.concatenate([W1[D:], zpad], axis=0).astype(jnp.bfloat16)

    outs = []
    for si in range(NSLICE):
        idx_s = lax.slice(idx, (si * nex_s * L,), ((si + 1) * nex_s * L,))
        prem_f, hyp_f = pool_kernel(emb_p, idx_s)
        outs.append(_mlp(prem_f, hyp_f, w1a, w1b, b1, W2, b2))
    return jnp.concatenate(outs)


# speedup vs baseline: 1.2641x; 1.1219x over previous
import functools

import jax
import jax.numpy as jnp
from jax import lax
from jax.experimental import pallas as pl
from jax.experimental.pallas import tpu as pltpu
from jax.experimental.pallas import tpu_sc as plsc

VOCAB = 100000
D = 100
DP = 128
L = 20
B = 16384
NEX = 2 * B
HIDDEN = 4096

CEX = 16
ROWS_PER_CHUNK = CEX * L
GATHERS = ((0, 128), (128, 128), (256, 64))


NSLICE = 4


def _make_sc_pool(nex_s):
    info = plsc.get_sparse_core_info()
    nc, ns = info.num_cores, info.num_subcores
    nw = nc * ns
    chunks_per_w = nex_s // nw // CEX
    pairs = chunks_per_w // 2
    idx_per_w = chunks_per_w * ROWS_PER_CHUNK

    mesh = plsc.VectorSubcoreMesh(core_axis_name="c", subcore_axis_name="s")

    @functools.partial(
        pl.kernel,
        mesh=mesh,
        out_type=(
            jax.ShapeDtypeStruct((nex_s // 2, DP), jnp.float32),
            jax.ShapeDtypeStruct((nex_s // 2, DP), jnp.float32),
        ),
        scratch_types=[
            pltpu.VMEM((idx_per_w,), jnp.int32),
            pltpu.VMEM((ROWS_PER_CHUNK, DP), jnp.float32),
            pltpu.VMEM((ROWS_PER_CHUNK, DP), jnp.float32),
            pltpu.VMEM((CEX // 2, DP), jnp.float32),
            pltpu.VMEM((CEX // 2, DP), jnp.float32),
            pltpu.VMEM((CEX // 2, DP), jnp.float32),
            pltpu.VMEM((CEX // 2, DP), jnp.float32),
            pltpu.SemaphoreType.DMA,
            pltpu.SemaphoreType.DMA,
            pltpu.SemaphoreType.DMA,
            pltpu.SemaphoreType.DMA,
        ],
    )
    def pool_kernel(table_hbm, idx_hbm, prem_hbm, hyp_hbm, idx_all,
                    rows_v0, rows_v1,
                    out_p0, out_p1, out_h0, out_h1,
                    sem0, sem1, osem0, osem1):
        wid = lax.axis_index("s") * nc + lax.axis_index("c")
        chunk0 = wid * chunks_per_w
        row_bufs = (rows_v0, rows_v1)
        outp_bufs = (out_p0, out_p1)
        outh_bufs = (out_h0, out_h1)
        sems = (sem0, sem1)
        osems = (osem0, osem1)

        pltpu.sync_copy(idx_hbm.at[pl.ds(wid * idx_per_w, idx_per_w)], idx_all)

        def issue(buf, c_local):
            base = c_local * ROWS_PER_CHUNK
            for off, gs in GATHERS:
                pltpu.async_copy(
                    table_hbm.at[idx_all.at[pl.ds(base + off, gs)]],
                    row_bufs[buf].at[pl.ds(off, gs)],
                    sems[buf],
                )

        def wait_buf(buf):
            pltpu.make_async_copy(
                table_hbm.at[pl.ds(0, ROWS_PER_CHUNK)],
                row_bufs[buf],
                sems[buf],
            ).wait()

        def wait_out(buf):
            pltpu.make_async_copy(
                outp_bufs[buf],
                prem_hbm.at[pl.ds(0, CEX // 2)],
                osems[buf],
            ).wait()
            pltpu.make_async_copy(
                outh_bufs[buf],
                prem_hbm.at[pl.ds(0, CEX // 2)],
                osems[buf],
            ).wait()

        def compute_store(buf, c_local, have_outstanding):
            rows_v = row_bufs[buf]
            out_p = outp_bufs[buf]
            out_h = outh_bufs[buf]

            @pl.when(have_outstanding)
            def _():
                wait_out(buf)

            def pair_ex_body(j, carry2):
                r0 = j * 2 * L
                for d in range(DP // 16):
                    sl = pl.ds(d * 16, 16)
                    acc = rows_v[r0, sl]
                    for l in range(1, L):
                        acc = jnp.maximum(acc, rows_v[r0 + l, sl])
                    out_p[j, sl] = acc
                    acch = rows_v[r0 + L, sl]
                    for l in range(1, L):
                        acch = jnp.maximum(acch, rows_v[r0 + L + l, sl])
                    out_h[j, sl] = acch
                return carry2

            lax.fori_loop(0, CEX // 2, pair_ex_body, 0, unroll=False)
            b0 = (chunk0 + c_local) * (CEX // 2)
            pltpu.async_copy(
                out_p, prem_hbm.at[pl.ds(b0, CEX // 2)], osems[buf])
            pltpu.async_copy(
                out_h, hyp_hbm.at[pl.ds(b0, CEX // 2)], osems[buf])

        issue(0, 0)

        def pair_body(t, carry):
            c0 = 2 * t
            issue(1, c0 + 1)
            wait_buf(0)
            compute_store(0, c0, t > 0)

            @pl.when(t < pairs - 1)
            def _():
                issue(0, c0 + 2)

            wait_buf(1)
            compute_store(1, c0 + 1, t > 0)
            return carry

        lax.fori_loop(0, pairs, pair_body, 0, unroll=False)
        wait_out(0)
        wait_out(1)

    return pool_kernel


_TR_BV = 2048
_TR_GRID = (VOCAB + _TR_BV - 1) // _TR_BV


def _transpose_body(xt_ref, o_ref):
    x = xt_ref[...]
    xp = jnp.concatenate(
        [x, jnp.zeros((DP - D, _TR_BV), jnp.float32)], axis=0)
    r = lax.broadcasted_iota(jnp.int32, (DP, DP), 0)
    c = lax.broadcasted_iota(jnp.int32, (DP, DP), 1)
    eye = jnp.where(r == c, 1.0, 0.0).astype(jnp.float32)
    o_ref[...] = lax.dot_general(
        xp, eye, (((0,), (0,)), ((), ())),
        preferred_element_type=jnp.float32)


def _transpose_pad(emb_t):
    return pl.pallas_call(
        _transpose_body,
        grid=(_TR_GRID,),
        in_specs=[pl.BlockSpec((D, _TR_BV), lambda i: (0, i))],
        out_specs=pl.BlockSpec((_TR_BV, DP), lambda i: (i, 0)),
        out_shape=jax.ShapeDtypeStruct((VOCAB, DP), jnp.float32),
    )(emb_t)


_TC_BM = 512


def _mlp_body(xp_ref, xh_ref, w1a_ref, w1b_ref, b1_ref, w2_ref, b2_ref, o_ref):
    xp = xp_ref[...].astype(jnp.bfloat16)
    xh = xh_ref[...].astype(jnp.bfloat16)
    h = jnp.dot(xp, w1a_ref[...], preferred_element_type=jnp.float32)
    h = h + jnp.dot(xh, w1b_ref[...], preferred_element_type=jnp.float32)
    h = jnp.maximum(h + b1_ref[...], 0.0)
    y = jnp.sum(h * w2_ref[...], axis=1) + b2_ref[0]
    o_ref[...] = jax.nn.sigmoid(y)


def _mlp(prem, hyp, w1a, w1b, b1, w2, b2):
    bs = prem.shape[0]
    grid = (bs // _TC_BM,)
    return pl.pallas_call(
        _mlp_body,
        grid=grid,
        in_specs=[
            pl.BlockSpec((_TC_BM, DP), lambda i: (i, 0)),
            pl.BlockSpec((_TC_BM, DP), lambda i: (i, 0)),
            pl.BlockSpec((DP, HIDDEN), lambda i: (0, 0)),
            pl.BlockSpec((DP, HIDDEN), lambda i: (0, 0)),
            pl.BlockSpec((1, HIDDEN), lambda i: (0, 0)),
            pl.BlockSpec((1, HIDDEN), lambda i: (0, 0)),
            pl.BlockSpec(memory_space=pltpu.SMEM),
        ],
        out_specs=pl.BlockSpec((_TC_BM,), lambda i: (i,)),
        out_shape=jax.ShapeDtypeStruct((bs,), jnp.float32),
    )(prem, hyp, w1a, w1b, b1.reshape(1, HIDDEN), w2.reshape(1, HIDDEN), b2)


def kernel(premise, hypothesis, emb_table, W1, b1, W2, b2):
    nex_s = NEX // NSLICE
    pool_kernel = _make_sc_pool(nex_s)

    emb_p = _transpose_pad(emb_table.T)

    zpad = jnp.zeros((DP - D, HIDDEN), dtype=W1.dtype)
    w1a = jnp.concatenate([W1[:D], zpad], axis=0).astype(jnp.bfloat16)
    w1b = jnp.concatenate([W1[D:], zpad], axis=0).astype(jnp.bfloat16)

    bsl = B // NSLICE
    outs = []
    for si in range(NSLICE):
        p_s = lax.slice(premise, (si * bsl, 0), ((si + 1) * bsl, L))
        h_s = lax.slice(hypothesis, (si * bsl, 0), ((si + 1) * bsl, L))
        idx_s = jnp.stack([p_s, h_s], axis=1).reshape(-1)
        prem_f, hyp_f = pool_kernel(emb_p, idx_s)
        outs.append(_mlp(prem_f, hyp_f, w1a, w1b, b1, W2, b2))
    return jnp.concatenate(outs)


# speedup vs baseline: 1.3934x; 1.1023x over previous
import functools

import jax
import jax.numpy as jnp
from jax import lax
from jax.experimental import pallas as pl
from jax.experimental.pallas import tpu as pltpu
from jax.experimental.pallas import tpu_sc as plsc

VOCAB = 100000
D = 100
DP = 128
L = 20
B = 16384
NEX = 2 * B
HIDDEN = 4096

CEX = 16
ROWS_PER_CHUNK = CEX * L
GATHERS = ((0, 128), (128, 128), (256, 64))


NSLICE = 4


def _make_sc_pool(nex_s):
    info = plsc.get_sparse_core_info()
    nc, ns = info.num_cores, info.num_subcores
    nw = nc * ns
    chunks_per_w = nex_s // nw // CEX
    pairs = chunks_per_w // 2
    idx_per_w = chunks_per_w * ROWS_PER_CHUNK

    mesh = plsc.VectorSubcoreMesh(core_axis_name="c", subcore_axis_name="s")

    @functools.partial(
        pl.kernel,
        mesh=mesh,
        out_type=(
            jax.ShapeDtypeStruct((nex_s // 2, DP), jnp.float32),
            jax.ShapeDtypeStruct((nex_s // 2, DP), jnp.float32),
        ),
        scratch_types=[
            pltpu.VMEM((idx_per_w,), jnp.int32),
            pltpu.VMEM((ROWS_PER_CHUNK, DP), jnp.float32),
            pltpu.VMEM((ROWS_PER_CHUNK, DP), jnp.float32),
            pltpu.VMEM((CEX // 2, DP), jnp.float32),
            pltpu.VMEM((CEX // 2, DP), jnp.float32),
            pltpu.VMEM((CEX // 2, DP), jnp.float32),
            pltpu.VMEM((CEX // 2, DP), jnp.float32),
            pltpu.SemaphoreType.DMA,
            pltpu.SemaphoreType.DMA,
            pltpu.SemaphoreType.DMA,
            pltpu.SemaphoreType.DMA,
        ],
    )
    def pool_kernel(table_hbm, idx_hbm, prem_hbm, hyp_hbm, idx_all,
                    rows_v0, rows_v1,
                    out_p0, out_p1, out_h0, out_h1,
                    sem0, sem1, osem0, osem1):
        wid = lax.axis_index("s") * nc + lax.axis_index("c")
        chunk0 = wid * chunks_per_w
        row_bufs = (rows_v0, rows_v1)
        outp_bufs = (out_p0, out_p1)
        outh_bufs = (out_h0, out_h1)
        sems = (sem0, sem1)
        osems = (osem0, osem1)

        pltpu.sync_copy(idx_hbm.at[pl.ds(wid * idx_per_w, idx_per_w)], idx_all)

        def issue(buf, c_local):
            base = c_local * ROWS_PER_CHUNK
            for off, gs in GATHERS:
                pltpu.async_copy(
                    table_hbm.at[idx_all.at[pl.ds(base + off, gs)]],
                    row_bufs[buf].at[pl.ds(off, gs)],
                    sems[buf],
                )

        def wait_buf(buf):
            pltpu.make_async_copy(
                table_hbm.at[pl.ds(0, ROWS_PER_CHUNK)],
                row_bufs[buf],
                sems[buf],
            ).wait()

        def wait_out(buf):
            pltpu.make_async_copy(
                outp_bufs[buf],
                prem_hbm.at[pl.ds(0, CEX // 2)],
                osems[buf],
            ).wait()
            pltpu.make_async_copy(
                outh_bufs[buf],
                prem_hbm.at[pl.ds(0, CEX // 2)],
                osems[buf],
            ).wait()

        def compute_store(buf, c_local, have_outstanding):
            rows_v = row_bufs[buf]
            out_p = outp_bufs[buf]
            out_h = outh_bufs[buf]

            @pl.when(have_outstanding)
            def _():
                wait_out(buf)

            def pair_ex_body(j, carry2):
                r0 = j * 2 * L
                for d in range(DP // 16):
                    sl = pl.ds(d * 16, 16)
                    acc = rows_v[r0, sl]
                    for l in range(1, L):
                        acc = jnp.maximum(acc, rows_v[r0 + l, sl])
                    out_p[j, sl] = acc
                    acch = rows_v[r0 + L, sl]
                    for l in range(1, L):
                        acch = jnp.maximum(acch, rows_v[r0 + L + l, sl])
                    out_h[j, sl] = acch
                return carry2

            lax.fori_loop(0, CEX // 2, pair_ex_body, 0, unroll=False)
            b0 = (chunk0 + c_local) * (CEX // 2)
            pltpu.async_copy(
                out_p, prem_hbm.at[pl.ds(b0, CEX // 2)], osems[buf])
            pltpu.async_copy(
                out_h, hyp_hbm.at[pl.ds(b0, CEX // 2)], osems[buf])

        issue(0, 0)

        def pair_body(t, carry):
            c0 = 2 * t
            issue(1, c0 + 1)
            wait_buf(0)
            compute_store(0, c0, t > 0)

            @pl.when(t < pairs - 1)
            def _():
                issue(0, c0 + 2)

            wait_buf(1)
            compute_store(1, c0 + 1, t > 0)
            return carry

        lax.fori_loop(0, pairs, pair_body, 0, unroll=False)
        wait_out(0)
        wait_out(1)

    return pool_kernel


_TR_BV = 4096
_TR_GRID = (VOCAB + _TR_BV - 1) // _TR_BV


def _transpose_body(xt_ref, o_ref):
    x = xt_ref[...]
    xp = jnp.concatenate(
        [x, jnp.zeros((DP - D, _TR_BV), jnp.float32)], axis=0)
    r = lax.broadcasted_iota(jnp.int32, (DP, DP), 0)
    c = lax.broadcasted_iota(jnp.int32, (DP, DP), 1)
    eye = jnp.where(r == c, 1.0, 0.0).astype(jnp.float32)
    o_ref[...] = lax.dot_general(
        xp, eye, (((0,), (0,)), ((), ())),
        preferred_element_type=jnp.float32)


def _transpose_pad(emb_t):
    return pl.pallas_call(
        _transpose_body,
        grid=(_TR_GRID,),
        in_specs=[pl.BlockSpec((D, _TR_BV), lambda i: (0, i))],
        out_specs=pl.BlockSpec((_TR_BV, DP), lambda i: (i, 0)),
        out_shape=jax.ShapeDtypeStruct((VOCAB, DP), jnp.float32),
    )(emb_t)


_TC_BM = 512


def _mlp_body(xp_ref, xh_ref, w1a_ref, w1b_ref, b1_ref, w2_ref, b2_ref, o_ref):
    xp = xp_ref[...].astype(jnp.bfloat16)
    xh = xh_ref[...].astype(jnp.bfloat16)
    h = jnp.dot(xp, w1a_ref[...], preferred_element_type=jnp.float32)
    h = h + jnp.dot(xh, w1b_ref[...], preferred_element_type=jnp.float32)
    h = jnp.maximum(h + b1_ref[...], 0.0)
    y = jnp.sum(h * w2_ref[...], axis=1) + b2_ref[0]
    o_ref[...] = jax.nn.sigmoid(y)


def _mlp(prem, hyp, w1a, w1b, b1, w2, b2):
    bs = prem.shape[0]
    grid = (bs // _TC_BM,)
    return pl.pallas_call(
        _mlp_body,
        grid=grid,
        in_specs=[
            pl.BlockSpec((_TC_BM, DP), lambda i: (i, 0)),
            pl.BlockSpec((_TC_BM, DP), lambda i: (i, 0)),
            pl.BlockSpec((DP, HIDDEN), lambda i: (0, 0)),
            pl.BlockSpec((DP, HIDDEN), lambda i: (0, 0)),
            pl.BlockSpec((1, HIDDEN), lambda i: (0, 0)),
            pl.BlockSpec((1, HIDDEN), lambda i: (0, 0)),
            pl.BlockSpec(memory_space=pltpu.SMEM),
        ],
        out_specs=pl.BlockSpec((_TC_BM,), lambda i: (i,)),
        out_shape=jax.ShapeDtypeStruct((bs,), jnp.float32),
    )(prem, hyp, w1a, w1b, b1.reshape(1, HIDDEN), w2.reshape(1, HIDDEN), b2)


def kernel(premise, hypothesis, emb_table, W1, b1, W2, b2):
    slice_b = (5120, 5120, 4096, 2048)

    emb_p = _transpose_pad(emb_table.T)

    zpad = jnp.zeros((DP - D, HIDDEN), dtype=W1.dtype)
    w1a = jnp.concatenate([W1[:D], zpad], axis=0).astype(jnp.bfloat16)
    w1b = jnp.concatenate([W1[D:], zpad], axis=0).astype(jnp.bfloat16)

    outs = []
    b0 = 0
    for bsl in slice_b:
        p_s = lax.slice(premise, (b0, 0), (b0 + bsl, L))
        h_s = lax.slice(hypothesis, (b0, 0), (b0 + bsl, L))
        idx_s = jnp.stack([p_s, h_s], axis=1).reshape(-1)
        prem_f, hyp_f = _make_sc_pool(2 * bsl)(emb_p, idx_s)
        outs.append(_mlp(prem_f, hyp_f, w1a, w1b, b1, W2, b2))
        b0 += bsl
    return jnp.concatenate(outs)
